# baseline (speedup 1.0000x reference)
import functools

import jax
import jax.numpy as jnp
from jax import lax
from jax.experimental import pallas as pl
from jax.experimental.pallas import tpu as pltpu
from jax.experimental.pallas import tpu_sc as plsc

_BATCH, _HIST, _D = 16384, 50, 64
_NC, _NS = 2, 16
_NW = _NC * _NS
_BW = _BATCH // _NW
_G = 128
_NT = _BW // _G
_NG = _HIST * _NT
_TA, _TJ = _D // 8, 8
_NBUF = 2


def _body(xt_hbm, tab_hbm, out_hbm, idx_v, rows0, rows1, tb0, tb1,
          gsems, ssems):
    rows_b = (rows0, rows1)
    tb_b = (tb0, tb1)

    wid = lax.axis_index("s") * _NC + lax.axis_index("c")
    bbase = wid * _BW
    cbase = wid * _NT

    pltpu.sync_copy(xt_hbm.at[:, pl.ds(bbase, _BW)], idx_v)

    iota = lax.iota(jnp.int32, 16)
    a_k = [(iota + 16 * k) // 8 for k in range(_D // 16)]
    jr_k = [(iota + 16 * k) % 8 for k in range(_D // 16)]

    def fire_gather(b, g):
        h = g // _NT
        ct = g % _NT
        pltpu.async_copy(
            tab_hbm.at[idx_v.at[h, pl.ds(ct * _G, _G)]],
            rows_b[b],
            gsems.at[b],
        )

    def drain_gather(b):
        pltpu.make_async_copy(
            tab_hbm.at[pl.ds(0, _G)], rows_b[b], gsems.at[b]
        ).wait()

    def scatter(b):
        @pl.loop(0, _G, unroll=8)
        def _row(r):
            br = jnp.full((16,), r, jnp.int32)
            for k in range(_D // 16):
                v = rows_b[b][r, pl.ds(16 * k, 16)]
                plsc.store_scatter(tb_b[b], [a_k[k], jr_k[k], br], v)

    def store(b, g):
        h = g // _NT
        ct = g % _NT
        pltpu.async_copy(
            tb_b[b],
            out_hbm.at[h, :, cbase + ct],
            ssems.at[b],
        )

    def wait_store(b):
        pltpu.make_async_copy(
            tb_b[b], out_hbm.at[0, :, 0], ssems.at[b]
        ).wait()

    fire_gather(0, 0)

    @pl.loop(0, _NG, step=_NBUF)
    def _outer(g0):
        for b in range(_NBUF):
            g = g0 + b
            nb = (b + 1) % _NBUF
            ng = g + 1

            @pl.when(ng < _NG)
            def _():
                fire_gather(nb, ng)

            drain_gather(b)

            @pl.when(g >= _NBUF)
            def _():
                wait_store(b)

            scatter(b)
            store(b, g)

    for b in range(_NBUF):
        wait_store(b)


@jax.jit
def _lookup(xt, img_weight):
    mesh = plsc.VectorSubcoreMesh(core_axis_name="c", subcore_axis_name="s")
    run = functools.partial(
        pl.kernel,
        out_type=jax.ShapeDtypeStruct(
            (_HIST, _TA, _BATCH // _G, _TJ, _G), jnp.float32),
        mesh=mesh,
        scratch_types=[
            pltpu.VMEM((_HIST, _BW), jnp.int32),
            pltpu.VMEM((_G, _D), jnp.float32),
            pltpu.VMEM((_G, _D), jnp.float32),
            pltpu.VMEM((_TA, _TJ, _G), jnp.float32),
            pltpu.VMEM((_TA, _TJ, _G), jnp.float32),
            pltpu.SemaphoreType.DMA((_NBUF,)),
            pltpu.SemaphoreType.DMA((_NBUF,)),
        ],
        compiler_params=pltpu.CompilerParams(
            use_tc_tiling_on_sc=False, needs_layout_passes=False),
    )(_body)
    return run(xt, img_weight)


def kernel(x, img_weight):
    five_d = _lookup(x.T, img_weight)
    out_t = lax.transpose(five_d, (2, 4, 0, 1, 3))
    return out_t.reshape(_BATCH, _HIST, _D)


# speedup vs baseline: 1.6182x; 1.6182x over previous
import functools

import jax
import jax.numpy as jnp
from jax import lax
from jax.experimental import pallas as pl
from jax.experimental.pallas import tpu as pltpu
from jax.experimental.pallas import tpu_sc as plsc

_BATCH, _HIST, _D = 16384, 50, 64
_NC, _NS = 2, 16
_NW = _NC * _NS
_BW = _BATCH // _NW
_G = 128
_NT = _BW // _G
_NG = _HIST * _NT
_TA, _TJ = _D // 8, 8
_NBUF = 2


def _body(xt_hbm, tab_hbm, out_hbm, idx_v, rows0, rows1, tb0, tb1,
          gsems, ssems):
    rows_b = (rows0, rows1)
    tb_b = (tb0, tb1)

    wid = lax.axis_index("s") * _NC + lax.axis_index("c")
    bbase = wid * _BW
    cbase = wid * _NT

    pltpu.sync_copy(xt_hbm.at[:, pl.ds(bbase, _BW)], idx_v)

    iota = lax.iota(jnp.int32, 16)
    a_k = [(iota + 16 * k) // 8 for k in range(_D // 16)]
    jr_k = [(iota + 16 * k) % 8 for k in range(_D // 16)]

    def fire_gather(b, g):
        h = g // _NT
        ct = g % _NT
        pltpu.async_copy(
            tab_hbm.at[idx_v.at[h, pl.ds(ct * _G, _G)]],
            rows_b[b],
            gsems.at[b],
        )

    def drain_gather(b):
        pltpu.make_async_copy(
            tab_hbm.at[pl.ds(0, _G)], rows_b[b], gsems.at[b]
        ).wait()

    def scatter(b):
        @pl.loop(0, _G, unroll=8)
        def _row(r):
            br = jnp.full((16,), r, jnp.int32)
            for k in range(_D // 16):
                v = rows_b[b][r, pl.ds(16 * k, 16)]
                plsc.store_scatter(tb_b[b], [a_k[k], jr_k[k], br], v)

    def store(b, g):
        h = g // _NT
        ct = g % _NT
        pltpu.async_copy(
            tb_b[b].at[:, :, pl.ds(0, _G)],
            out_hbm.at[h, :, cbase + ct],
            ssems.at[b],
        )

    def wait_store(b):
        pltpu.make_async_copy(
            tb_b[b].at[:, :, pl.ds(0, _G)], out_hbm.at[0, :, 0], ssems.at[b]
        ).wait()

    fire_gather(0, 0)

    @pl.loop(0, _NG, step=_NBUF)
    def _outer(g0):
        for b in range(_NBUF):
            g = g0 + b
            nb = (b + 1) % _NBUF
            ng = g + 1

            @pl.when(ng < _NG)
            def _():
                fire_gather(nb, ng)

            drain_gather(b)

            @pl.when(g >= _NBUF)
            def _():
                wait_store(b)

            scatter(b)
            store(b, g)

    for b in range(_NBUF):
        wait_store(b)


@jax.jit
def _lookup(xt, img_weight):
    mesh = plsc.VectorSubcoreMesh(core_axis_name="c", subcore_axis_name="s")
    run = functools.partial(
        pl.kernel,
        out_type=jax.ShapeDtypeStruct(
            (_HIST, _TA, _BATCH // _G, _TJ, _G), jnp.float32),
        mesh=mesh,
        scratch_types=[
            pltpu.VMEM((_HIST, _BW), jnp.int32),
            pltpu.VMEM((_G, _D), jnp.float32),
            pltpu.VMEM((_G, _D), jnp.float32),
            pltpu.VMEM((_TA, _TJ, _G + 1), jnp.float32),
            pltpu.VMEM((_TA, _TJ, _G + 1), jnp.float32),
            pltpu.SemaphoreType.DMA((_NBUF,)),
            pltpu.SemaphoreType.DMA((_NBUF,)),
        ],
        compiler_params=pltpu.CompilerParams(
            use_tc_tiling_on_sc=False, needs_layout_passes=False),
    )(_body)
    return run(xt, img_weight)


def kernel(x, img_weight):
    five_d = _lookup(x.T, img_weight)
    out_t = lax.transpose(five_d, (2, 4, 0, 1, 3))
    return out_t.reshape(_BATCH, _HIST, _D)


# speedup vs baseline: 1.8034x; 1.1144x over previous
import functools

import jax
import jax.numpy as jnp
from jax import lax
from jax.experimental import pallas as pl
from jax.experimental.pallas import tpu as pltpu
from jax.experimental.pallas import tpu_sc as plsc

_BATCH, _HIST, _D = 16384, 50, 64
_NC, _NS = 2, 16
_NW = _NC * _NS
_BW = _BATCH // _NW
_G = 128
_NT = _BW // _G
_NG = _HIST * _NT
_TA, _TJ = _D // 8, 8
_NBUF = 2


def _body(xt_hbm, tab_hbm, out_hbm, idx_v, rows0, rows1, tb0, tb1,
          gsems, ssems):
    rows_b = (rows0, rows1)
    tb_b = (tb0, tb1)

    wid = lax.axis_index("s") * _NC + lax.axis_index("c")
    bbase = wid * _BW
    cbase = wid * _NT

    pltpu.sync_copy(xt_hbm.at[:, pl.ds(bbase, _BW)], idx_v)

    iota = lax.iota(jnp.int32, 16)
    a_k = [(iota + 16 * k) // 8 for k in range(_D // 16)]
    jr_k = [(iota + 16 * k) % 8 for k in range(_D // 16)]

    def fire_gather(b, g):
        h = g // _NT
        ct = g % _NT
        pltpu.async_copy(
            tab_hbm.at[idx_v.at[h, pl.ds(ct * _G, _G)]],
            rows_b[b],
            gsems.at[b],
        )

    def drain_gather(b):
        pltpu.make_async_copy(
            tab_hbm.at[pl.ds(0, _G)], rows_b[b], gsems.at[b]
        ).wait()

    def scatter(b):
        @pl.loop(0, _G, unroll=8)
        def _row(r):
            br = jnp.full((16,), r, jnp.int32)
            vs = [rows_b[b][r, pl.ds(16 * k, 16)] for k in range(_D // 16)]
            for k in range(_D // 16):
                plsc.store_scatter(tb_b[b], [a_k[k], jr_k[k], br], vs[k])

    def store(b, g):
        h = g // _NT
        ct = g % _NT
        pltpu.async_copy(
            tb_b[b].at[:, :, pl.ds(0, _G)],
            out_hbm.at[h, :, cbase + ct],
            ssems.at[b],
        )

    def wait_store(b):
        pltpu.make_async_copy(
            tb_b[b].at[:, :, pl.ds(0, _G)], out_hbm.at[0, :, 0], ssems.at[b]
        ).wait()

    fire_gather(0, 0)

    @pl.loop(0, _NG, step=_NBUF)
    def _outer(g0):
        for b in range(_NBUF):
            g = g0 + b
            nb = (b + 1) % _NBUF
            ng = g + 1

            @pl.when(ng < _NG)
            def _():
                fire_gather(nb, ng)

            drain_gather(b)

            @pl.when(g >= _NBUF)
            def _():
                wait_store(b)

            scatter(b)
            store(b, g)

    for b in range(_NBUF):
        wait_store(b)


@jax.jit
def _lookup(xt, img_weight):
    mesh = plsc.VectorSubcoreMesh(core_axis_name="c", subcore_axis_name="s")
    run = functools.partial(
        pl.kernel,
        out_type=jax.ShapeDtypeStruct(
            (_HIST, _TA, _BATCH // _G, _TJ, _G), jnp.float32),
        mesh=mesh,
        scratch_types=[
            pltpu.VMEM((_HIST, _BW), jnp.int32),
            pltpu.VMEM((_G, _D), jnp.float32),
            pltpu.VMEM((_G, _D), jnp.float32),
            pltpu.VMEM((_TA, _TJ, _G + 1), jnp.float32),
            pltpu.VMEM((_TA, _TJ, _G + 1), jnp.float32),
            pltpu.SemaphoreType.DMA((_NBUF,)),
            pltpu.SemaphoreType.DMA((_NBUF,)),
        ],
        compiler_params=pltpu.CompilerParams(
            use_tc_tiling_on_sc=False, needs_layout_passes=False),
    )(_body)
    return run(xt, img_weight)


def kernel(x, img_weight):
    five_d = _lookup(x.T, img_weight)
    out_t = lax.transpose(five_d, (2, 4, 0, 1, 3))
    return out_t.reshape(_BATCH, _HIST, _D)
